# Initial kernel scaffold; baseline (speedup 1.0000x reference)
import functools

import jax
import jax.numpy as jnp
from jax import lax
from jax.experimental import pallas as pl
from jax.experimental.pallas import tpu as pltpu
from jax.experimental.pallas import tpu_sc as plsc

N_NODES = 10000
D = 128
E = 320000

NC = 2
NS = 16
NW = NC * NS
NPAD = 10240
EPT = E // NW
G = 80
NG = EPT // G
RPT = NPAD // NS
ZR = 80

_MESH = plsc.VectorSubcoreMesh(core_axis_name="c", subcore_axis_name="s")


@functools.partial(
    pl.kernel,
    out_type=jax.ShapeDtypeStruct((NC, NPAD), jnp.float32),
    mesh=_MESH,
    scratch_types=[
        pltpu.VMEM((NG, G), jnp.int32),
        pltpu.VMEM((NPAD,), jnp.float32),
        pltpu.VMEM((NS, RPT), jnp.float32),
        pltpu.VMEM_SHARED((NS, NPAD), jnp.float32),
    ],
)
def _deg_kernel(dst_h, parts_h, dstv, hist, rbuf, shared):
  cid = lax.axis_index("c")
  sid = lax.axis_index("s")
  wid = cid * NS + sid
  zero16 = jnp.zeros((16,), jnp.float32)
  ones16 = jnp.full((16,), 1.0, jnp.float32)

  def zh(i, carry):
    hist[pl.ds(i * 16, 16)] = zero16
    return carry

  lax.fori_loop(0, NPAD // 16, zh, 0)
  pltpu.sync_copy(dst_h.at[wid], dstv)

  def e_body(g, carry):
    for j in range(G // 16):
      idx = dstv[g, pl.ds(j * 16, 16)]
      plsc.addupdate_scatter(hist, [idx], ones16)
    return carry

  lax.fori_loop(0, NG, e_body, 0)

  pltpu.sync_copy(hist, shared.at[sid])
  plsc.subcore_barrier()
  pltpu.sync_copy(shared.at[:, pl.ds(sid * RPT, RPT)], rbuf)

  def r_body(cv, carry):
    a = rbuf[0, pl.ds(cv * 16, 16)]
    for r in range(1, NS):
      a = a + rbuf[r, pl.ds(cv * 16, 16)]
    hist[pl.ds(cv * 16, 16)] = a
    return carry

  lax.fori_loop(0, RPT // 16, r_body, 0)
  pltpu.sync_copy(hist.at[pl.ds(0, RPT)], parts_h.at[cid, pl.ds(sid * RPT, RPT)])


def _make_prop(with_scalar):
  out_type = [jax.ShapeDtypeStruct((NC, NPAD, D), jnp.float32)]
  scratch = [
      pltpu.VMEM((NG, G), jnp.int32),
      pltpu.VMEM((NG, G), jnp.int32),
      pltpu.VMEM((G, D), jnp.float32),
      pltpu.VMEM((ZR, D), jnp.float32),
      pltpu.VMEM_SHARED((NPAD, D), jnp.float32),
  ]
  if with_scalar:
    out_type.append(jax.ShapeDtypeStruct((NC, NPAD), jnp.float32))
    scratch += [
        pltpu.VMEM((NPAD,), jnp.float32),
        pltpu.VMEM((NPAD,), jnp.float32),
        pltpu.VMEM((NS, RPT), jnp.float32),
        pltpu.VMEM_SHARED((NS, NPAD), jnp.float32),
    ]

  def body(*refs):
    if with_scalar:
      (src_h, dst_h, table_h, dvec_h, parts_h, sparts_h,
       srcv, dstv, rows, zbuf, acc, dvecv, hist, rbuf, shared_s) = refs
    else:
      (src_h, dst_h, table_h, parts_h,
       srcv, dstv, rows, zbuf, acc) = refs
    cid = lax.axis_index("c")
    sid = lax.axis_index("s")
    wid = cid * NS + sid
    zero16 = jnp.zeros((16,), jnp.float32)

    def zb(i, carry):
      for j in range(D // 16):
        zbuf[i, pl.ds(j * 16, 16)] = zero16
      return carry

    lax.fori_loop(0, ZR, zb, 0)
    for k in range(RPT // ZR):
      pltpu.sync_copy(zbuf, acc.at[pl.ds(sid * RPT + k * ZR, ZR)])
    if with_scalar:
      def zh(i, carry):
        hist[pl.ds(i * 16, 16)] = zero16
        return carry

      lax.fori_loop(0, NPAD // 16, zh, 0)
      pltpu.sync_copy(dvec_h, dvecv)
    pltpu.sync_copy(src_h.at[wid], srcv)
    pltpu.sync_copy(dst_h.at[wid], dstv)
    plsc.subcore_barrier()

    def e_body(g, carry):
      pltpu.sync_copy(table_h.at[srcv.at[g]], rows)
      pltpu.sync_copy(rows, acc.at[dstv.at[g]], add=True)
      if with_scalar:
        for j in range(G // 16):
          sidx = srcv[g, pl.ds(j * 16, 16)]
          didx = dstv[g, pl.ds(j * 16, 16)]
          vals = plsc.load_gather(dvecv, [sidx])
          plsc.addupdate_scatter(hist, [didx], vals)
      return carry

    lax.fori_loop(0, NG, e_body, 0)
    plsc.subcore_barrier()
    pltpu.sync_copy(acc.at[pl.ds(sid * RPT, RPT)],
                    parts_h.at[cid, pl.ds(sid * RPT, RPT)])
    if with_scalar:
      pltpu.sync_copy(hist, shared_s.at[sid])
      plsc.subcore_barrier()
      pltpu.sync_copy(shared_s.at[:, pl.ds(sid * RPT, RPT)], rbuf)

      def r_body(cv, carry):
        a = rbuf[0, pl.ds(cv * 16, 16)]
        for r in range(1, NS):
          a = a + rbuf[r, pl.ds(cv * 16, 16)]
        hist[pl.ds(cv * 16, 16)] = a
        return carry

      lax.fori_loop(0, RPT // 16, r_body, 0)
      pltpu.sync_copy(hist.at[pl.ds(0, RPT)],
                      sparts_h.at[cid, pl.ds(sid * RPT, RPT)])

  return functools.partial(
      pl.kernel, out_type=out_type, mesh=_MESH, scratch_types=scratch)(body)


_prop_scalar = _make_prop(True)
_prop_plain = _make_prop(False)


_RB = 1024


def _prep_body(degp_ref, emb_ref, dinv_ref, p0_ref):
  deg = degp_ref[0] + degp_ref[1] + 1.0
  dinv = lax.rsqrt(deg)
  dinv_ref[...] = dinv
  p0_ref[...] = emb_ref[...] * dinv[:, None]


def _prep_call(deg_parts, embp):
  return pl.pallas_call(
      _prep_body,
      grid=(NPAD // _RB,),
      in_specs=[
          pl.BlockSpec((NC, _RB), lambda i: (0, i)),
          pl.BlockSpec((_RB, D), lambda i: (i, 0)),
      ],
      out_specs=[
          pl.BlockSpec((_RB,), lambda i: (i,)),
          pl.BlockSpec((_RB, D), lambda i: (i, 0)),
      ],
      out_shape=[
          jax.ShapeDtypeStruct((NPAD,), jnp.float32),
          jax.ShapeDtypeStruct((NPAD, D), jnp.float32),
      ],
  )(deg_parts, embp)


def _mid_body(parts_ref, p0_ref, dinv_ref, p1_ref):
  d = dinv_ref[...]
  t = parts_ref[0] + parts_ref[1] + p0_ref[...]
  p1_ref[...] = t * (d * d)[:, None]


def _mid_call(parts0, p0, dinv):
  return pl.pallas_call(
      _mid_body,
      grid=(NPAD // _RB,),
      in_specs=[
          pl.BlockSpec((NC, _RB, D), lambda i: (0, i, 0)),
          pl.BlockSpec((_RB, D), lambda i: (i, 0)),
          pl.BlockSpec((_RB,), lambda i: (i,)),
      ],
      out_specs=pl.BlockSpec((_RB, D), lambda i: (i, 0)),
      out_shape=jax.ShapeDtypeStruct((NPAD, D), jnp.float32),
  )(parts0, p0, dinv)


def _final_body(parts_ref, p1_ref, dinv_ref, sparts_ref,
                w1_ref, b1_ref, w2_ref, b2_ref, out_ref):
  d = dinv_ref[...]
  q2 = (parts_ref[0] + parts_ref[1] + p1_ref[...]) * d[:, None]
  w12 = jnp.dot(w1_ref[...], w2_ref[...], preferred_element_type=jnp.float32)
  b12 = jnp.dot(b1_ref[...][None, :], w2_ref[...],
                preferred_element_type=jnp.float32)[0]
  s = d * (sparts_ref[0] + sparts_ref[1] + d)
  out_ref[...] = (jnp.dot(q2, w12, preferred_element_type=jnp.float32)
                  + s[:, None] * b12[None, :] + b2_ref[...][None, :])


def _final_call(parts1, p1, dinv, sparts, W1, b1, W2, b2):
  return pl.pallas_call(
      _final_body,
      grid=(NPAD // _RB,),
      in_specs=[
          pl.BlockSpec((NC, _RB, D), lambda i: (0, i, 0)),
          pl.BlockSpec((_RB, D), lambda i: (i, 0)),
          pl.BlockSpec((_RB,), lambda i: (i,)),
          pl.BlockSpec((NC, _RB), lambda i: (0, i)),
          pl.BlockSpec((D, 2 * D), lambda i: (0, 0)),
          pl.BlockSpec((2 * D,), lambda i: (0,)),
          pl.BlockSpec((2 * D, D), lambda i: (0, 0)),
          pl.BlockSpec((D,), lambda i: (0,)),
      ],
      out_specs=pl.BlockSpec((_RB, D), lambda i: (i, 0)),
      out_shape=jax.ShapeDtypeStruct((NPAD, D), jnp.float32),
  )(parts1, p1, dinv, sparts, W1, b1, W2, b2)


def kernel(edge_index, edge_weight, emb, W1, b1, W2, b2):
  src = edge_index[0].astype(jnp.int32)
  dst = edge_index[1].astype(jnp.int32)
  src3 = src.reshape(NW, NG, G)
  dst3 = dst.reshape(NW, NG, G)
  embp = jnp.pad(emb, ((0, NPAD - N_NODES), (0, 0)))

  deg_parts = _deg_kernel(dst3)
  dinv, p0 = _prep_call(deg_parts, embp)
  parts0, sparts = _prop_scalar(src3, dst3, p0, dinv)
  p1 = _mid_call(parts0, p0, dinv)
  parts1 = _prop_plain(src3, dst3, p1)
  out = _final_call(parts1, p1, dinv, sparts, W1, b1, W2, b2)
  return out[:N_NODES]


# speedup vs baseline: 14.0444x; 14.0444x over previous
import functools

import jax
import jax.numpy as jnp
from jax import lax
from jax.experimental import pallas as pl
from jax.experimental.pallas import tpu as pltpu
from jax.experimental.pallas import tpu_sc as plsc

N_NODES = 10000
D = 128
E = 320000

NC = 2
NS = 16
NW = NC * NS
NPAD = 10240
EPT = E // NW
G = 80
NG = EPT // G
RPT = NPAD // NS
ZR = 80

_MESH = plsc.VectorSubcoreMesh(core_axis_name="c", subcore_axis_name="s")
_SC_PARAMS = pltpu.CompilerParams(needs_layout_passes=False,
                                  use_tc_tiling_on_sc=False)


@functools.partial(
    pl.kernel,
    out_type=jax.ShapeDtypeStruct((NC, NPAD), jnp.float32),
    mesh=_MESH,
    compiler_params=_SC_PARAMS,
    scratch_types=[
        pltpu.VMEM((NG, G), jnp.int32),
        pltpu.VMEM((NPAD,), jnp.float32),
        pltpu.VMEM((NS, RPT), jnp.float32),
        pltpu.VMEM_SHARED((NS, NPAD), jnp.float32),
    ],
)
def _deg_kernel(dst_h, parts_h, dstv, hist, rbuf, shared):
  cid = lax.axis_index("c")
  sid = lax.axis_index("s")
  wid = cid * NS + sid
  zero16 = jnp.zeros((16,), jnp.float32)
  ones16 = jnp.full((16,), 1.0, jnp.float32)

  def zh(i, carry):
    hist[pl.ds(i * 16, 16)] = zero16
    return carry

  lax.fori_loop(0, NPAD // 16, zh, 0)
  pltpu.sync_copy(dst_h.at[wid], dstv)

  def e_body(g, carry):
    for j in range(G // 16):
      idx = dstv[g, pl.ds(j * 16, 16)]
      plsc.addupdate_scatter(hist, [idx], ones16)
    return carry

  lax.fori_loop(0, NG, e_body, 0)

  pltpu.sync_copy(hist, shared.at[sid])
  plsc.subcore_barrier()
  pltpu.sync_copy(shared.at[:, pl.ds(sid * RPT, RPT)], rbuf)

  def r_body(cv, carry):
    a = rbuf[0, pl.ds(cv * 16, 16)]
    for r in range(1, NS):
      a = a + rbuf[r, pl.ds(cv * 16, 16)]
    hist[pl.ds(cv * 16, 16)] = a
    return carry

  lax.fori_loop(0, RPT // 16, r_body, 0)
  pltpu.sync_copy(hist.at[pl.ds(0, RPT)], parts_h.at[cid, pl.ds(sid * RPT, RPT)])


DH = D // 2


def _make_prop(with_scalar):
  out_type = [jax.ShapeDtypeStruct((NC, NPAD, 2, DH), jnp.float32)]
  scratch = [
      pltpu.VMEM((NG, G), jnp.int32),
      pltpu.VMEM((NG, G), jnp.int32),
      pltpu.VMEM((G, DH), jnp.float32),
      pltpu.VMEM((ZR, DH), jnp.float32),
      pltpu.VMEM_SHARED((NPAD, DH), jnp.float32),
  ]
  if with_scalar:
    out_type.append(jax.ShapeDtypeStruct((NW, NPAD), jnp.float32))
    scratch += [
        pltpu.VMEM((NPAD,), jnp.float32),
        pltpu.VMEM((NPAD,), jnp.float32),
    ]

  def body(*refs):
    if with_scalar:
      (srcs_h, dst_h, table_h, dvec_h, parts_h, sparts_h,
       srcv, dstv, rows, zbuf, acc, dvecv, hist) = refs
    else:
      (srcs_h, dst_h, table_h, parts_h,
       srcv, dstv, rows, zbuf, acc) = refs
    cid = lax.axis_index("c")
    sid = lax.axis_index("s")
    wid = cid * NS + sid
    zero16 = jnp.zeros((16,), jnp.float32)

    def zb(i, carry):
      for j in range(DH // 16):
        zbuf[i, pl.ds(j * 16, 16)] = zero16
      return carry

    lax.fori_loop(0, ZR, zb, 0)
    for k in range(RPT // ZR):
      pltpu.sync_copy(zbuf, acc.at[pl.ds(sid * RPT + k * ZR, ZR)])
    if with_scalar:
      def zh(i, carry):
        hist[pl.ds(i * 16, 16)] = zero16
        return carry

      lax.fori_loop(0, NPAD // 16, zh, 0)
      pltpu.sync_copy(dvec_h, dvecv)
    pltpu.sync_copy(dst_h.at[wid], dstv)

    for h in range(2):
      pltpu.sync_copy(srcs_h.at[h, wid], srcv)
      plsc.subcore_barrier()

      def e_body(g, carry):
        pltpu.sync_copy(table_h.at[srcv.at[g]], rows)
        pltpu.sync_copy(rows, acc.at[dstv.at[g]], add=True)
        if with_scalar and h == 0:
          for j in range(G // 16):
            sidx = lax.shift_right_logical(srcv[g, pl.ds(j * 16, 16)], 1)
            didx = dstv[g, pl.ds(j * 16, 16)]
            vals = plsc.load_gather(dvecv, [sidx])
            plsc.addupdate_scatter(hist, [didx], vals)
        return carry

      lax.fori_loop(0, NG, e_body, 0)
      plsc.subcore_barrier()
      pltpu.sync_copy(acc.at[pl.ds(sid * RPT, RPT)],
                      parts_h.at[cid, pl.ds(sid * RPT, RPT), h])
      if h == 0:
        for k in range(RPT // ZR):
          pltpu.sync_copy(zbuf, acc.at[pl.ds(sid * RPT + k * ZR, ZR)])
    if with_scalar:
      pltpu.sync_copy(hist, sparts_h.at[wid])

  return functools.partial(
      pl.kernel, out_type=out_type, mesh=_MESH, scratch_types=scratch,
      compiler_params=_SC_PARAMS)(body)


_prop_scalar = _make_prop(True)
_prop_plain = _make_prop(False)


_RB = 1024


def _prep_body(degp_ref, emb_ref, dinv_ref, p0_ref):
  deg = degp_ref[0] + degp_ref[1] + 1.0
  dinv = lax.rsqrt(deg)
  dinv_ref[...] = dinv
  p0_ref[...] = emb_ref[...] * dinv[:, None]


def _prep_call(deg_parts, embp):
  return pl.pallas_call(
      _prep_body,
      grid=(NPAD // _RB,),
      in_specs=[
          pl.BlockSpec((NC, _RB), lambda i: (0, i)),
          pl.BlockSpec((_RB, D), lambda i: (i, 0)),
      ],
      out_specs=[
          pl.BlockSpec((_RB,), lambda i: (i,)),
          pl.BlockSpec((_RB, D), lambda i: (i, 0)),
      ],
      out_shape=[
          jax.ShapeDtypeStruct((NPAD,), jnp.float32),
          jax.ShapeDtypeStruct((NPAD, D), jnp.float32),
      ],
  )(deg_parts, embp)


def _mid_body(parts_ref, p0_ref, dinv_ref, p1_ref):
  d = dinv_ref[...]
  t = parts_ref[0] + parts_ref[1] + p0_ref[...]
  p1_ref[...] = t * (d * d)[:, None]


def _mid_call(parts0, p0, dinv):
  return pl.pallas_call(
      _mid_body,
      grid=(NPAD // _RB,),
      in_specs=[
          pl.BlockSpec((NC, _RB, D), lambda i: (0, i, 0)),
          pl.BlockSpec((_RB, D), lambda i: (i, 0)),
          pl.BlockSpec((_RB,), lambda i: (i,)),
      ],
      out_specs=pl.BlockSpec((_RB, D), lambda i: (i, 0)),
      out_shape=jax.ShapeDtypeStruct((NPAD, D), jnp.float32),
  )(parts0, p0, dinv)


def _final_body(parts_ref, p1_ref, dinv_ref, sparts_ref,
                w1_ref, b1_ref, w2_ref, b2_ref, out_ref):
  d = dinv_ref[...]
  q2 = (parts_ref[0] + parts_ref[1] + p1_ref[...]) * d[:, None]
  w12 = jnp.dot(w1_ref[...], w2_ref[...], preferred_element_type=jnp.float32)
  b12 = jnp.dot(b1_ref[...][None, :], w2_ref[...],
                preferred_element_type=jnp.float32)[0]
  s = d * (jnp.sum(sparts_ref[...], axis=0) + d)
  out_ref[...] = (jnp.dot(q2, w12, preferred_element_type=jnp.float32)
                  + s[:, None] * b12[None, :] + b2_ref[...][None, :])


def _final_call(parts1, p1, dinv, sparts, W1, b1, W2, b2):
  return pl.pallas_call(
      _final_body,
      grid=(NPAD // _RB,),
      in_specs=[
          pl.BlockSpec((NC, _RB, D), lambda i: (0, i, 0)),
          pl.BlockSpec((_RB, D), lambda i: (i, 0)),
          pl.BlockSpec((_RB,), lambda i: (i,)),
          pl.BlockSpec((NW, _RB), lambda i: (0, i)),
          pl.BlockSpec((D, 2 * D), lambda i: (0, 0)),
          pl.BlockSpec((2 * D,), lambda i: (0,)),
          pl.BlockSpec((2 * D, D), lambda i: (0, 0)),
          pl.BlockSpec((D,), lambda i: (0,)),
      ],
      out_specs=pl.BlockSpec((_RB, D), lambda i: (i, 0)),
      out_shape=jax.ShapeDtypeStruct((NPAD, D), jnp.float32),
  )(parts1, p1, dinv, sparts, W1, b1, W2, b2)


def kernel(edge_index, edge_weight, emb, W1, b1, W2, b2):
  src = edge_index[0].astype(jnp.int32)
  dst = edge_index[1].astype(jnp.int32)
  src2 = src * 2
  srcs = jnp.stack([src2, src2 + 1]).reshape(2, NW, NG, G)
  dst3 = dst.reshape(NW, NG, G)
  embp = jnp.pad(emb, ((0, NPAD - N_NODES), (0, 0)))

  deg_parts = _deg_kernel(dst3)
  dinv, p0 = _prep_call(deg_parts, embp)
  parts0, sparts = _prop_scalar(srcs, dst3, p0.reshape(2 * NPAD, DH), dinv)
  p1 = _mid_call(parts0.reshape(NC, NPAD, D), p0, dinv)
  (parts1,) = _prop_plain(srcs, dst3, p1.reshape(2 * NPAD, DH))
  out = _final_call(parts1.reshape(NC, NPAD, D), p1, dinv, sparts,
                    W1, b1, W2, b2)
  return out[:N_NODES]


# speedup vs baseline: 20.6918x; 1.4733x over previous
import functools

import jax
import jax.numpy as jnp
from jax import lax
from jax.experimental import pallas as pl
from jax.experimental.pallas import tpu as pltpu
from jax.experimental.pallas import tpu_sc as plsc

N_NODES = 10000
D = 128
E = 320000

NC = 2
NS = 16
NW = NC * NS
NPAD = 10240
EPT = E // NW
G = 80
NG = EPT // G
RPT = NPAD // NS
ZR = 80

_MESH = plsc.VectorSubcoreMesh(core_axis_name="c", subcore_axis_name="s")
_SC_PARAMS = pltpu.CompilerParams(needs_layout_passes=False,
                                  use_tc_tiling_on_sc=False)


@functools.partial(
    pl.kernel,
    out_type=jax.ShapeDtypeStruct((NC, NPAD), jnp.float32),
    mesh=_MESH,
    compiler_params=_SC_PARAMS,
    scratch_types=[
        pltpu.VMEM((NG, G), jnp.int32),
        pltpu.VMEM((NPAD,), jnp.float32),
        pltpu.VMEM((NS, RPT), jnp.float32),
        pltpu.VMEM_SHARED((NS, NPAD), jnp.float32),
    ],
)
def _deg_kernel(dst_h, parts_h, dstv, hist, rbuf, shared):
  cid = lax.axis_index("c")
  sid = lax.axis_index("s")
  wid = cid * NS + sid
  zero16 = jnp.zeros((16,), jnp.float32)
  ones16 = jnp.full((16,), 1.0, jnp.float32)

  def zh(i, carry):
    hist[pl.ds(i * 16, 16)] = zero16
    return carry

  lax.fori_loop(0, NPAD // 16, zh, 0)
  pltpu.sync_copy(dst_h.at[wid], dstv)

  def e_body(g, carry):
    for j in range(G // 16):
      idx = dstv[g, pl.ds(j * 16, 16)]
      plsc.addupdate_scatter(hist, [idx], ones16)
    return carry

  lax.fori_loop(0, NG, e_body, 0)

  pltpu.sync_copy(hist, shared.at[sid])
  plsc.subcore_barrier()
  pltpu.sync_copy(shared.at[:, pl.ds(sid * RPT, RPT)], rbuf)

  def r_body(cv, carry):
    a = rbuf[0, pl.ds(cv * 16, 16)]
    for r in range(1, NS):
      a = a + rbuf[r, pl.ds(cv * 16, 16)]
    hist[pl.ds(cv * 16, 16)] = a
    return carry

  lax.fori_loop(0, RPT // 16, r_body, 0)
  pltpu.sync_copy(hist.at[pl.ds(0, RPT)], parts_h.at[cid, pl.ds(sid * RPT, RPT)])


DH = D // 2


def _make_prop(with_scalar):
  out_type = [jax.ShapeDtypeStruct((NC, NPAD, 2, DH), jnp.float32)]
  scratch = [
      pltpu.VMEM((NG, G), jnp.int32),
      pltpu.VMEM((NG, G), jnp.int32),
      pltpu.VMEM((G, DH), jnp.float32),
      pltpu.VMEM((G, DH), jnp.float32),
      pltpu.VMEM((ZR, DH), jnp.float32),
      pltpu.VMEM_SHARED((NPAD, DH), jnp.float32),
      pltpu.SemaphoreType.DMA,
      pltpu.SemaphoreType.DMA,
  ]
  if with_scalar:
    out_type.append(jax.ShapeDtypeStruct((NW, NPAD), jnp.float32))
    scratch += [
        pltpu.VMEM((NPAD,), jnp.float32),
        pltpu.VMEM((NPAD,), jnp.float32),
    ]

  def body(*refs):
    if with_scalar:
      (srcs_h, dst_h, table_h, dvec_h, parts_h, sparts_h,
       srcv, dstv, rows0, rows1, zbuf, acc, sem0, sem1, dvecv, hist) = refs
    else:
      (srcs_h, dst_h, table_h, parts_h,
       srcv, dstv, rows0, rows1, zbuf, acc, sem0, sem1) = refs
    cid = lax.axis_index("c")
    sid = lax.axis_index("s")
    wid = cid * NS + sid
    zero16 = jnp.zeros((16,), jnp.float32)

    def zb(i, carry):
      for j in range(DH // 16):
        zbuf[i, pl.ds(j * 16, 16)] = zero16
      return carry

    lax.fori_loop(0, ZR, zb, 0)
    for k in range(RPT // ZR):
      pltpu.sync_copy(zbuf, acc.at[pl.ds(sid * RPT + k * ZR, ZR)])
    if with_scalar:
      def zh(i, carry):
        hist[pl.ds(i * 16, 16)] = zero16
        return carry

      lax.fori_loop(0, NPAD // 16, zh, 0)
      pltpu.sync_copy(dvec_h, dvecv)
    pltpu.sync_copy(dst_h.at[wid], dstv)

    for h in range(2):
      pltpu.sync_copy(srcs_h.at[h, wid], srcv)
      plsc.subcore_barrier()

      do_scalar = with_scalar and h == 0

      def scalar_work(g):
        for j in range(G // 16):
          sidx = lax.shift_right_logical(srcv[g, pl.ds(j * 16, 16)], 1)
          didx = dstv[g, pl.ds(j * 16, 16)]
          vals = plsc.load_gather(dvecv, [sidx])
          plsc.addupdate_scatter(hist, [didx], vals)

      def wait_gather(rows, sem):
        pltpu.make_async_copy(table_h.at[pl.ds(0, G)], rows, sem).wait()

      pltpu.async_copy(table_h.at[srcv.at[0]], rows0, sem0)

      def pair_body(gg, carry):
        g0 = 2 * gg
        pltpu.async_copy(table_h.at[srcv.at[g0 + 1]], rows1, sem1)
        if do_scalar:
          scalar_work(g0)
        wait_gather(rows0, sem0)
        pltpu.sync_copy(rows0, acc.at[dstv.at[g0]], add=True)
        pltpu.async_copy(table_h.at[srcv.at[g0 + 2]], rows0, sem0)
        if do_scalar:
          scalar_work(g0 + 1)
        wait_gather(rows1, sem1)
        pltpu.sync_copy(rows1, acc.at[dstv.at[g0 + 1]], add=True)
        return carry

      lax.fori_loop(0, (NG - 1) // 2, pair_body, 0)
      if do_scalar:
        scalar_work(NG - 1)
      wait_gather(rows0, sem0)
      pltpu.sync_copy(rows0, acc.at[dstv.at[NG - 1]], add=True)
      plsc.subcore_barrier()
      pltpu.sync_copy(acc.at[pl.ds(sid * RPT, RPT)],
                      parts_h.at[cid, pl.ds(sid * RPT, RPT), h])
      if h == 0:
        for k in range(RPT // ZR):
          pltpu.sync_copy(zbuf, acc.at[pl.ds(sid * RPT + k * ZR, ZR)])
    if with_scalar:
      pltpu.sync_copy(hist, sparts_h.at[wid])

  return functools.partial(
      pl.kernel, out_type=out_type, mesh=_MESH, scratch_types=scratch,
      compiler_params=_SC_PARAMS)(body)


_prop_scalar = _make_prop(True)
_prop_plain = _make_prop(False)


_RB = 1024


def _prep_body(degp_ref, emb_ref, dinv_ref, p0_ref):
  deg = degp_ref[0] + degp_ref[1] + 1.0
  dinv = lax.rsqrt(deg)
  dinv_ref[...] = dinv
  p0_ref[...] = emb_ref[...] * dinv[:, None]


def _prep_call(deg_parts, embp):
  return pl.pallas_call(
      _prep_body,
      grid=(NPAD // _RB,),
      in_specs=[
          pl.BlockSpec((NC, _RB), lambda i: (0, i)),
          pl.BlockSpec((_RB, D), lambda i: (i, 0)),
      ],
      out_specs=[
          pl.BlockSpec((_RB,), lambda i: (i,)),
          pl.BlockSpec((_RB, D), lambda i: (i, 0)),
      ],
      out_shape=[
          jax.ShapeDtypeStruct((NPAD,), jnp.float32),
          jax.ShapeDtypeStruct((NPAD, D), jnp.float32),
      ],
  )(deg_parts, embp)


def _mid_body(parts_ref, p0_ref, dinv_ref, p1_ref):
  d = dinv_ref[...]
  t = parts_ref[0] + parts_ref[1] + p0_ref[...]
  p1_ref[...] = t * (d * d)[:, None]


def _mid_call(parts0, p0, dinv):
  return pl.pallas_call(
      _mid_body,
      grid=(NPAD // _RB,),
      in_specs=[
          pl.BlockSpec((NC, _RB, D), lambda i: (0, i, 0)),
          pl.BlockSpec((_RB, D), lambda i: (i, 0)),
          pl.BlockSpec((_RB,), lambda i: (i,)),
      ],
      out_specs=pl.BlockSpec((_RB, D), lambda i: (i, 0)),
      out_shape=jax.ShapeDtypeStruct((NPAD, D), jnp.float32),
  )(parts0, p0, dinv)


def _final_body(parts_ref, p1_ref, dinv_ref, sparts_ref,
                w1_ref, b1_ref, w2_ref, b2_ref, out_ref):
  d = dinv_ref[...]
  q2 = (parts_ref[0] + parts_ref[1] + p1_ref[...]) * d[:, None]
  w12 = jnp.dot(w1_ref[...], w2_ref[...], preferred_element_type=jnp.float32)
  b12 = jnp.dot(b1_ref[...][None, :], w2_ref[...],
                preferred_element_type=jnp.float32)[0]
  s = d * (jnp.sum(sparts_ref[...], axis=0) + d)
  out_ref[...] = (jnp.dot(q2, w12, preferred_element_type=jnp.float32)
                  + s[:, None] * b12[None, :] + b2_ref[...][None, :])


def _final_call(parts1, p1, dinv, sparts, W1, b1, W2, b2):
  return pl.pallas_call(
      _final_body,
      grid=(NPAD // _RB,),
      in_specs=[
          pl.BlockSpec((NC, _RB, D), lambda i: (0, i, 0)),
          pl.BlockSpec((_RB, D), lambda i: (i, 0)),
          pl.BlockSpec((_RB,), lambda i: (i,)),
          pl.BlockSpec((NW, _RB), lambda i: (0, i)),
          pl.BlockSpec((D, 2 * D), lambda i: (0, 0)),
          pl.BlockSpec((2 * D,), lambda i: (0,)),
          pl.BlockSpec((2 * D, D), lambda i: (0, 0)),
          pl.BlockSpec((D,), lambda i: (0,)),
      ],
      out_specs=pl.BlockSpec((_RB, D), lambda i: (i, 0)),
      out_shape=jax.ShapeDtypeStruct((NPAD, D), jnp.float32),
  )(parts1, p1, dinv, sparts, W1, b1, W2, b2)


def kernel(edge_index, edge_weight, emb, W1, b1, W2, b2):
  src = edge_index[0].astype(jnp.int32)
  dst = edge_index[1].astype(jnp.int32)
  src2 = src * 2
  srcs = jnp.stack([src2, src2 + 1]).reshape(2, NW, NG, G)
  dst3 = dst.reshape(NW, NG, G)
  embp = jnp.pad(emb, ((0, NPAD - N_NODES), (0, 0)))

  deg_parts = _deg_kernel(dst3)
  dinv, p0 = _prep_call(deg_parts, embp)
  parts0, sparts = _prop_scalar(srcs, dst3, p0.reshape(2 * NPAD, DH), dinv)
  p1 = _mid_call(parts0.reshape(NC, NPAD, D), p0, dinv)
  (parts1,) = _prop_plain(srcs, dst3, p1.reshape(2 * NPAD, DH))
  out = _final_call(parts1.reshape(NC, NPAD, D), p1, dinv, sparts,
                    W1, b1, W2, b2)
  return out[:N_NODES]
